# baseline (speedup 1.0000x reference)
import functools

import jax
import jax.numpy as jnp
from jax import lax
from jax.experimental import pallas as pl
from jax.experimental.pallas import tpu as pltpu
from jax.experimental.pallas import tpu_sc as plsc

N = 16384
C = 256
K = 8192
R = 256
KC = 2048
WIN = 2736

NW = 32
BPW = N // NW
CH = 128
NCH = BPW // CH


def _argmin_body(x2_ref, e2_ref, x_ref, emb_ref, idx_ref, d_ref):
    nwin = -(-K // WIN)
    wmin = [None] * nwin
    iic = lax.broadcasted_iota(jnp.int32, (R, KC), 1)
    for c in range(K // KC):
        lo = c * KC
        hi = lo + KC
        embc = emb_ref[pl.ds(lo, KC), :]
        mm = lax.dot_general(
            x_ref[:], embc,
            dimension_numbers=(((1,), (1,)), ((), ())),
            preferred_element_type=jnp.float32,
        )
        d = (x2_ref[:] + e2_ref[0:1, pl.ds(lo, KC)]) - 2.0 * mm
        d_ref[:, pl.ds(lo, KC)] = d
        for w in range(nwin):
            ws, we = w * WIN, min((w + 1) * WIN, K)
            if we <= lo or ws >= hi:
                continue
            if ws <= lo and we >= hi:
                m = jnp.min(d, axis=1, keepdims=True)
            else:
                seg = (iic >= ws - lo) & (iic < we - lo)
                m = jnp.min(jnp.where(seg, d, jnp.inf), axis=1, keepdims=True)
            wmin[w] = m if wmin[w] is None else jnp.minimum(wmin[w], m)

    acc = wmin[0].astype(jnp.bfloat16).astype(jnp.float32)
    vexact = wmin[0]
    win = jnp.zeros((R, 1), jnp.int32)
    for w in range(1, nwin):
        take = wmin[w] < acc
        vexact = jnp.where(take, wmin[w], vexact)
        win = jnp.where(take, w, win)
        acc = jnp.where(take, wmin[w], acc).astype(jnp.bfloat16).astype(jnp.float32)

    ii = lax.broadcasted_iota(jnp.int32, (R, K), 1)
    ws = win * WIN
    d = d_ref[:]
    hit = (d == vexact) & (ii >= ws) & (ii < ws + WIN)
    idx_ref[:] = jnp.min(jnp.where(hit, ii, K), axis=1, keepdims=True)


_argmin_call = pl.pallas_call(
    _argmin_body,
    grid=(N // R,),
    in_specs=[
        pl.BlockSpec((R, 1), lambda i: (i, 0)),
        pl.BlockSpec((1, K), lambda i: (0, 0)),
        pl.BlockSpec((R, C), lambda i: (i, 0)),
        pl.BlockSpec((K, C), lambda i: (0, 0)),
    ],
    out_specs=pl.BlockSpec((R, 1), lambda i: (i, 0)),
    out_shape=jax.ShapeDtypeStruct((N, 1), jnp.int32),
    scratch_shapes=[pltpu.VMEM((R, K), jnp.float32)],
    compiler_params=pltpu.CompilerParams(
        dimension_semantics=("arbitrary",),
    ),
)


@functools.lru_cache(maxsize=1)
def _make_gather_kernel():
    nc = plsc.get_sparse_core_info().num_cores

    @functools.partial(
        pl.kernel,
        out_type=jax.ShapeDtypeStruct((N, C), jnp.float32),
        mesh=plsc.VectorSubcoreMesh(core_axis_name="c", subcore_axis_name="s"),
        scratch_types=[
            pltpu.VMEM((NCH, CH), jnp.int32),
            pltpu.VMEM((CH, C), jnp.float32),
            pltpu.VMEM((CH, C), jnp.float32),
            pltpu.SemaphoreType.DMA,
            pltpu.SemaphoreType.DMA,
        ],
    )
    def gather_kernel(idx_hbm, emb_hbm, out_hbm, idx_v, buf0, buf1, sem0, sem1):
        wid = lax.axis_index("s") * nc + lax.axis_index("c")
        base = wid * BPW
        pltpu.sync_copy(idx_hbm.at[wid], idx_v)
        bufs = (buf0, buf1)
        sems = (sem0, sem1)
        cp = pltpu.async_copy(emb_hbm.at[idx_v.at[0]], buf0, sem0)
        for j in range(NCH):
            cp.wait()
            if j + 1 < NCH:
                cp = pltpu.async_copy(
                    emb_hbm.at[idx_v.at[j + 1]], bufs[(j + 1) % 2], sems[(j + 1) % 2]
                )
            pltpu.sync_copy(bufs[j % 2], out_hbm.at[pl.ds(base + j * CH, CH)])

    return gather_kernel


def kernel(x, emb):
    B, _, H, W = x.shape
    x_flat = jnp.transpose(x, (0, 2, 3, 1)).reshape(-1, C)
    x2 = jnp.sum(x_flat ** 2, axis=1, keepdims=True)
    e2 = jnp.sum(emb ** 2, axis=1, keepdims=True).T
    idx = _argmin_call(x2, e2, x_flat, emb)
    zq_flat = _make_gather_kernel()(idx.reshape(NW, NCH, CH), emb)
    z_q = jnp.transpose(zq_flat.reshape(B, H, W, C), (0, 3, 1, 2))
    return (x, z_q)


# speedup vs baseline: 1.2009x; 1.2009x over previous
import functools

import jax
import jax.numpy as jnp
from jax import lax
from jax.experimental import pallas as pl
from jax.experimental.pallas import tpu as pltpu
from jax.experimental.pallas import tpu_sc as plsc

N = 16384
C = 256
K = 8192
R = 256
KC = 2048
WIN = 2736

NW = 32
BPW = N // NW
CH = 128
NCH = BPW // CH


def _argmin_body(x2_ref, e2_ref, x_ref, emb_ref, idx_ref):
    nwin = -(-K // WIN)
    wv = [None] * nwin
    wi = [None] * nwin
    iic = lax.broadcasted_iota(jnp.int32, (R, KC), 1)
    for c in range(K // KC):
        lo = c * KC
        hi = lo + KC
        embc = emb_ref[pl.ds(lo, KC), :]
        mm = lax.dot_general(
            x_ref[:], embc,
            dimension_numbers=(((1,), (1,)), ((), ())),
            preferred_element_type=jnp.float32,
        )
        d = (x2_ref[:] + e2_ref[0:1, pl.ds(lo, KC)]) - 2.0 * mm
        for w in range(nwin):
            ws, we = w * WIN, min((w + 1) * WIN, K)
            if we <= lo or ws >= hi:
                continue
            if ws <= lo and we >= hi:
                dm = d
            else:
                dm = jnp.where((iic >= ws - lo) & (iic < we - lo), d, jnp.inf)
            cmin = jnp.min(dm, axis=1, keepdims=True)
            cidx = jnp.min(
                jnp.where(dm == cmin, iic, K), axis=1, keepdims=True
            ) + lo
            if wv[w] is None:
                wv[w], wi[w] = cmin, cidx
            else:
                take = cmin < wv[w]
                wi[w] = jnp.where(take, cidx, wi[w])
                wv[w] = jnp.where(take, cmin, wv[w])

    acc = wv[0].astype(jnp.bfloat16).astype(jnp.float32)
    bidx = wi[0]
    for w in range(1, nwin):
        take = wv[w] < acc
        bidx = jnp.where(take, wi[w], bidx)
        acc = jnp.where(take, wv[w], acc).astype(jnp.bfloat16).astype(jnp.float32)
    idx_ref[:] = bidx


_argmin_call = pl.pallas_call(
    _argmin_body,
    grid=(N // R,),
    in_specs=[
        pl.BlockSpec((R, 1), lambda i: (i, 0)),
        pl.BlockSpec((1, K), lambda i: (0, 0)),
        pl.BlockSpec((R, C), lambda i: (i, 0)),
        pl.BlockSpec((K, C), lambda i: (0, 0)),
    ],
    out_specs=pl.BlockSpec((R, 1), lambda i: (i, 0)),
    out_shape=jax.ShapeDtypeStruct((N, 1), jnp.int32),
    compiler_params=pltpu.CompilerParams(
        dimension_semantics=("arbitrary",),
    ),
)


@functools.lru_cache(maxsize=1)
def _make_gather_kernel():
    nc = plsc.get_sparse_core_info().num_cores

    @functools.partial(
        pl.kernel,
        out_type=jax.ShapeDtypeStruct((N, C), jnp.float32),
        mesh=plsc.VectorSubcoreMesh(core_axis_name="c", subcore_axis_name="s"),
        scratch_types=[
            pltpu.VMEM((NCH, CH), jnp.int32),
            pltpu.VMEM((CH, C), jnp.float32),
            pltpu.VMEM((CH, C), jnp.float32),
            pltpu.SemaphoreType.DMA,
            pltpu.SemaphoreType.DMA,
        ],
    )
    def gather_kernel(idx_hbm, emb_hbm, out_hbm, idx_v, buf0, buf1, sem0, sem1):
        wid = lax.axis_index("s") * nc + lax.axis_index("c")
        base = wid * BPW
        pltpu.sync_copy(idx_hbm.at[wid], idx_v)
        bufs = (buf0, buf1)
        sems = (sem0, sem1)
        cp = pltpu.async_copy(emb_hbm.at[idx_v.at[0]], buf0, sem0)
        for j in range(NCH):
            cp.wait()
            if j + 1 < NCH:
                cp = pltpu.async_copy(
                    emb_hbm.at[idx_v.at[j + 1]], bufs[(j + 1) % 2], sems[(j + 1) % 2]
                )
            pltpu.sync_copy(bufs[j % 2], out_hbm.at[pl.ds(base + j * CH, CH)])

    return gather_kernel


def kernel(x, emb):
    B, _, H, W = x.shape
    x_flat = jnp.transpose(x, (0, 2, 3, 1)).reshape(-1, C)
    x2 = jnp.sum(x_flat ** 2, axis=1, keepdims=True)
    e2 = jnp.sum(emb ** 2, axis=1, keepdims=True).T
    idx = _argmin_call(x2, e2, x_flat, emb)
    zq_flat = _make_gather_kernel()(idx.reshape(NW, NCH, CH), emb)
    z_q = jnp.transpose(zq_flat.reshape(B, H, W, C), (0, 3, 1, 2))
    return (x, z_q)


# speedup vs baseline: 1.2738x; 1.0608x over previous
import functools

import jax
import jax.numpy as jnp
from jax import lax
from jax.experimental import pallas as pl
from jax.experimental.pallas import tpu as pltpu
from jax.experimental.pallas import tpu_sc as plsc

N = 16384
C = 256
K = 8192
R = 512
KC = 2048
WIN = 2736

NW = 32
BPW = N // NW
CH = 128
NCH = BPW // CH


def _argmin_body(x2_ref, e2_ref, x_ref, emb_ref, idx_ref):
    nwin = -(-K // WIN)
    wv = [None] * nwin
    wi = [None] * nwin
    iic = lax.broadcasted_iota(jnp.int32, (R, KC), 1)
    for c in range(K // KC):
        lo = c * KC
        hi = lo + KC
        embc = emb_ref[pl.ds(lo, KC), :]
        mm = lax.dot_general(
            x_ref[:], embc,
            dimension_numbers=(((1,), (1,)), ((), ())),
            preferred_element_type=jnp.float32,
        )
        d = (x2_ref[:] + e2_ref[0:1, pl.ds(lo, KC)]) - 2.0 * mm
        for w in range(nwin):
            ws, we = w * WIN, min((w + 1) * WIN, K)
            if we <= lo or ws >= hi:
                continue
            if ws <= lo and we >= hi:
                dm = d
            else:
                dm = jnp.where((iic >= ws - lo) & (iic < we - lo), d, jnp.inf)
            cmin = jnp.min(dm, axis=1, keepdims=True)
            cidx = jnp.min(
                jnp.where(dm == cmin, iic, K), axis=1, keepdims=True
            ) + lo
            if wv[w] is None:
                wv[w], wi[w] = cmin, cidx
            else:
                take = cmin < wv[w]
                wi[w] = jnp.where(take, cidx, wi[w])
                wv[w] = jnp.where(take, cmin, wv[w])

    acc = wv[0].astype(jnp.bfloat16).astype(jnp.float32)
    bidx = wi[0]
    for w in range(1, nwin):
        take = wv[w] < acc
        bidx = jnp.where(take, wi[w], bidx)
        acc = jnp.where(take, wv[w], acc).astype(jnp.bfloat16).astype(jnp.float32)
    idx_ref[:] = bidx


_argmin_call = pl.pallas_call(
    _argmin_body,
    grid=(N // R,),
    in_specs=[
        pl.BlockSpec((R, 1), lambda i: (i, 0)),
        pl.BlockSpec((1, K), lambda i: (0, 0)),
        pl.BlockSpec((R, C), lambda i: (i, 0)),
        pl.BlockSpec((K, C), lambda i: (0, 0)),
    ],
    out_specs=pl.BlockSpec((R, 1), lambda i: (i, 0)),
    out_shape=jax.ShapeDtypeStruct((N, 1), jnp.int32),
    compiler_params=pltpu.CompilerParams(
        dimension_semantics=("arbitrary",),
    ),
)


@functools.lru_cache(maxsize=1)
def _make_gather_kernel():
    nc = plsc.get_sparse_core_info().num_cores

    @functools.partial(
        pl.kernel,
        out_type=jax.ShapeDtypeStruct((N, C), jnp.float32),
        mesh=plsc.VectorSubcoreMesh(core_axis_name="c", subcore_axis_name="s"),
        scratch_types=[
            pltpu.VMEM((NCH, CH), jnp.int32),
            pltpu.VMEM((CH, C), jnp.float32),
            pltpu.VMEM((CH, C), jnp.float32),
            pltpu.SemaphoreType.DMA,
            pltpu.SemaphoreType.DMA,
        ],
    )
    def gather_kernel(idx_hbm, emb_hbm, out_hbm, idx_v, buf0, buf1, sem0, sem1):
        wid = lax.axis_index("s") * nc + lax.axis_index("c")
        base = wid * BPW
        pltpu.sync_copy(idx_hbm.at[wid], idx_v)
        bufs = (buf0, buf1)
        sems = (sem0, sem1)
        cp = pltpu.async_copy(emb_hbm.at[idx_v.at[0]], buf0, sem0)
        for j in range(NCH):
            cp.wait()
            if j + 1 < NCH:
                cp = pltpu.async_copy(
                    emb_hbm.at[idx_v.at[j + 1]], bufs[(j + 1) % 2], sems[(j + 1) % 2]
                )
            pltpu.sync_copy(bufs[j % 2], out_hbm.at[pl.ds(base + j * CH, CH)])

    return gather_kernel


def kernel(x, emb):
    B, _, H, W = x.shape
    x_flat = jnp.transpose(x, (0, 2, 3, 1)).reshape(-1, C)
    x2 = jnp.sum(x_flat ** 2, axis=1, keepdims=True)
    e2 = jnp.sum(emb ** 2, axis=1, keepdims=True).T
    idx = _argmin_call(x2, e2, x_flat, emb)
    zq_flat = _make_gather_kernel()(idx.reshape(NW, NCH, CH), emb)
    z_q = jnp.transpose(zq_flat.reshape(B, H, W, C), (0, 3, 1, 2))
    return (x, z_q)


# speedup vs baseline: 1.3562x; 1.0646x over previous
import functools

import jax
import jax.numpy as jnp
from jax import lax
from jax.experimental import pallas as pl
from jax.experimental.pallas import tpu as pltpu
from jax.experimental.pallas import tpu_sc as plsc

N = 16384
C = 256
K = 8192
R = 1024
KC = 2048
WIN = 2736

NW = 32
BPW = N // NW
CH = 128
NCH = BPW // CH


def _argmin_body(x2_ref, e2_ref, x_ref, emb_ref, idx_ref):
    nwin = -(-K // WIN)
    wv = [None] * nwin
    wi = [None] * nwin
    iic = lax.broadcasted_iota(jnp.int32, (R, KC), 1)
    for c in range(K // KC):
        lo = c * KC
        hi = lo + KC
        embc = emb_ref[pl.ds(lo, KC), :]
        mm = lax.dot_general(
            x_ref[:], embc,
            dimension_numbers=(((1,), (1,)), ((), ())),
            preferred_element_type=jnp.float32,
        )
        d = (x2_ref[:] + e2_ref[0:1, pl.ds(lo, KC)]) - mm
        for w in range(nwin):
            ws, we = w * WIN, min((w + 1) * WIN, K)
            if we <= lo or ws >= hi:
                continue
            if ws <= lo and we >= hi:
                dm = d
            else:
                dm = jnp.where((iic >= ws - lo) & (iic < we - lo), d, jnp.inf)
            cmin = jnp.min(dm, axis=1, keepdims=True)
            cidx = jnp.min(
                jnp.where(dm == cmin, iic, K), axis=1, keepdims=True
            ) + lo
            if wv[w] is None:
                wv[w], wi[w] = cmin, cidx
            else:
                take = cmin < wv[w]
                wi[w] = jnp.where(take, cidx, wi[w])
                wv[w] = jnp.where(take, cmin, wv[w])

    acc = wv[0].astype(jnp.bfloat16).astype(jnp.float32)
    bidx = wi[0]
    for w in range(1, nwin):
        take = wv[w] < acc
        bidx = jnp.where(take, wi[w], bidx)
        acc = jnp.where(take, wv[w], acc).astype(jnp.bfloat16).astype(jnp.float32)
    idx_ref[:] = bidx


_argmin_call = pl.pallas_call(
    _argmin_body,
    grid=(N // R,),
    in_specs=[
        pl.BlockSpec((R, 1), lambda i: (i, 0)),
        pl.BlockSpec((1, K), lambda i: (0, 0)),
        pl.BlockSpec((R, C), lambda i: (i, 0)),
        pl.BlockSpec((K, C), lambda i: (0, 0)),
    ],
    out_specs=pl.BlockSpec((R, 1), lambda i: (i, 0)),
    out_shape=jax.ShapeDtypeStruct((N, 1), jnp.int32),
    compiler_params=pltpu.CompilerParams(
        dimension_semantics=("arbitrary",),
    ),
)


@functools.lru_cache(maxsize=1)
def _make_gather_kernel():
    nc = plsc.get_sparse_core_info().num_cores

    @functools.partial(
        pl.kernel,
        out_type=jax.ShapeDtypeStruct((N, C), jnp.float32),
        mesh=plsc.VectorSubcoreMesh(core_axis_name="c", subcore_axis_name="s"),
        scratch_types=[
            pltpu.VMEM((NCH, CH), jnp.int32),
            pltpu.VMEM((CH, C), jnp.float32),
            pltpu.VMEM((CH, C), jnp.float32),
            pltpu.SemaphoreType.DMA,
            pltpu.SemaphoreType.DMA,
        ],
    )
    def gather_kernel(idx_hbm, emb_hbm, out_hbm, idx_v, buf0, buf1, sem0, sem1):
        wid = lax.axis_index("s") * nc + lax.axis_index("c")
        base = wid * BPW
        pltpu.sync_copy(idx_hbm.at[wid], idx_v)
        bufs = (buf0, buf1)
        sems = (sem0, sem1)
        cp = pltpu.async_copy(emb_hbm.at[idx_v.at[0]], buf0, sem0)
        for j in range(NCH):
            cp.wait()
            if j + 1 < NCH:
                cp = pltpu.async_copy(
                    emb_hbm.at[idx_v.at[j + 1]], bufs[(j + 1) % 2], sems[(j + 1) % 2]
                )
            pltpu.sync_copy(bufs[j % 2], out_hbm.at[pl.ds(base + j * CH, CH)])

    return gather_kernel


def kernel(x, emb):
    B, _, H, W = x.shape
    x_flat = jnp.transpose(x, (0, 2, 3, 1)).reshape(-1, C)
    x2 = jnp.sum(x_flat ** 2, axis=1, keepdims=True)
    e2 = jnp.sum(emb ** 2, axis=1, keepdims=True).T
    idx = _argmin_call(x2, e2, 2.0 * x_flat, emb)
    zq_flat = _make_gather_kernel()(idx.reshape(NW, NCH, CH), emb)
    z_q = jnp.transpose(zq_flat.reshape(B, H, W, C), (0, 3, 1, 2))
    return (x, z_q)


# speedup vs baseline: 1.8039x; 1.3301x over previous
import functools

import jax
import jax.numpy as jnp
from jax import lax
from jax.experimental import pallas as pl
from jax.experimental.pallas import tpu as pltpu
from jax.experimental.pallas import tpu_sc as plsc

N = 16384
C = 256
K = 8192
R = 1024
KC = 2048
WIN = 2736

NW = 32
BPW = N // NW
CH = 128
NCH = BPW // CH


def _argmin_body(x2_ref, e2_ref, x_ref, emb_ref, idx_ref):
    nwin = -(-K // WIN)
    L = 128
    lane = lax.broadcasted_iota(jnp.int32, (R, L), 1)
    wv = [jnp.full((R, L), jnp.inf, jnp.float32) for _ in range(nwin)]
    wi = [jnp.full((R, L), K, jnp.int32) for _ in range(nwin)]
    for c in range(K // KC):
        lo = c * KC
        embc = emb_ref[pl.ds(lo, KC), :]
        mm = lax.dot_general(
            x_ref[:], embc,
            dimension_numbers=(((1,), (1,)), ((), ())),
            preferred_element_type=jnp.float32,
        )
        d = (x2_ref[:] + e2_ref[0:1, pl.ds(lo, KC)]) - mm
        for j in range(KC // L):
            g0 = lo + j * L
            d_j = d[:, j * L:(j + 1) * L]
            col = lane + g0
            w_start = g0 // WIN
            w_end = (g0 + L - 1) // WIN
            for w in range(w_start, w_end + 1):
                take = d_j < wv[w]
                if w_start != w_end:
                    take &= (col >= w * WIN) & (col < (w + 1) * WIN)
                wi[w] = jnp.where(take, col, wi[w])
                wv[w] = jnp.where(take, d_j, wv[w])

    acc = None
    bidx = None
    for w in range(nwin):
        cmin = jnp.min(wv[w], axis=1, keepdims=True)
        cidx = jnp.min(jnp.where(wv[w] == cmin, wi[w], K), axis=1, keepdims=True)
        if acc is None:
            acc = cmin.astype(jnp.bfloat16).astype(jnp.float32)
            bidx = cidx
        else:
            take = cmin < acc
            bidx = jnp.where(take, cidx, bidx)
            acc = jnp.where(take, cmin, acc).astype(jnp.bfloat16).astype(jnp.float32)
    idx_ref[:] = bidx


_argmin_call = pl.pallas_call(
    _argmin_body,
    grid=(N // R,),
    in_specs=[
        pl.BlockSpec((R, 1), lambda i: (i, 0)),
        pl.BlockSpec((1, K), lambda i: (0, 0)),
        pl.BlockSpec((R, C), lambda i: (i, 0)),
        pl.BlockSpec((K, C), lambda i: (0, 0)),
    ],
    out_specs=pl.BlockSpec((R, 1), lambda i: (i, 0)),
    out_shape=jax.ShapeDtypeStruct((N, 1), jnp.int32),
    compiler_params=pltpu.CompilerParams(
        dimension_semantics=("arbitrary",),
    ),
)


@functools.lru_cache(maxsize=1)
def _make_gather_kernel():
    nc = plsc.get_sparse_core_info().num_cores

    @functools.partial(
        pl.kernel,
        out_type=jax.ShapeDtypeStruct((N, C), jnp.float32),
        mesh=plsc.VectorSubcoreMesh(core_axis_name="c", subcore_axis_name="s"),
        scratch_types=[
            pltpu.VMEM((NCH, CH), jnp.int32),
            pltpu.VMEM((CH, C), jnp.float32),
            pltpu.VMEM((CH, C), jnp.float32),
            pltpu.SemaphoreType.DMA,
            pltpu.SemaphoreType.DMA,
        ],
    )
    def gather_kernel(idx_hbm, emb_hbm, out_hbm, idx_v, buf0, buf1, sem0, sem1):
        wid = lax.axis_index("s") * nc + lax.axis_index("c")
        base = wid * BPW
        pltpu.sync_copy(idx_hbm.at[wid], idx_v)
        bufs = (buf0, buf1)
        sems = (sem0, sem1)
        cp = pltpu.async_copy(emb_hbm.at[idx_v.at[0]], buf0, sem0)
        for j in range(NCH):
            cp.wait()
            if j + 1 < NCH:
                cp = pltpu.async_copy(
                    emb_hbm.at[idx_v.at[j + 1]], bufs[(j + 1) % 2], sems[(j + 1) % 2]
                )
            pltpu.sync_copy(bufs[j % 2], out_hbm.at[pl.ds(base + j * CH, CH)])

    return gather_kernel


def kernel(x, emb):
    B, _, H, W = x.shape
    x_flat = jnp.transpose(x, (0, 2, 3, 1)).reshape(-1, C)
    x2 = jnp.sum(x_flat ** 2, axis=1, keepdims=True)
    e2 = jnp.sum(emb ** 2, axis=1, keepdims=True).T
    idx = _argmin_call(x2, e2, 2.0 * x_flat, emb)
    zq_flat = _make_gather_kernel()(idx.reshape(NW, NCH, CH), emb)
    z_q = jnp.transpose(zq_flat.reshape(B, H, W, C), (0, 3, 1, 2))
    return (x, z_q)
